# baseline (speedup 1.0000x reference)
import functools

import jax
import jax.numpy as jnp
from jax import lax
from jax.experimental import pallas as pl
from jax.experimental.pallas import tpu as pltpu, tpu_sc as plsc

B = 16384
D = 128
G = 8
N_GENRES = 100
T_ROWS = 1001

NC = 2
NS = 16
NW = NC * NS
BPW = B // NW
L = 16
NCHUNK = BPW // 128
GRPS = 128 // L


def _sc_body(t_idx_hbm, genres_hbm, t_table_hbm, style_hbm, out_hbm,
             idx_v, gv_v, style_v, rows_s, t_buf0, t_buf1,
             sem0, sem1, semw, sems):
    wid = lax.axis_index("s") * NC + lax.axis_index("c")
    base = wid * BPW

    pltpu.sync_copy(t_idx_hbm.at[pl.ds(wid * NCHUNK, NCHUNK), :], idx_v)
    pltpu.sync_copy(genres_hbm.at[pl.ds(base * (G // 2), BPW * G // 2)], gv_v)
    pltpu.sync_copy(style_hbm, style_v)

    t_bufs = [t_buf0, t_buf1]
    gsems = [sem0, sem1]
    iota = lax.iota(jnp.int32, L)

    gathers = [None] * NCHUNK
    writes = [None] * NCHUNK
    gathers[0] = pltpu.async_copy(t_table_hbm.at[idx_v.at[0]], t_buf0, sem0)

    for j in range(NCHUNK):
        if j >= 1:
            writes[j - 1].wait()
        if j + 1 < NCHUNK:
            gathers[j + 1] = pltpu.async_copy(
                t_table_hbm.at[idx_v.at[j + 1]], t_bufs[(j + 1) % 2],
                gsems[(j + 1) % 2])

        @plsc.parallel_loop(j * 128, (j + 1) * 128, unroll=1)
        def _elem(i):
            sb = jnp.full((L,), i * (G // 2), jnp.int32)
            words = [plsc.load_gather(gv_v, [sb + k]) for k in range(G // 2)]
            vg = []
            for w in words:
                vg.append(((w & 0x7F) << 7) + iota)
                vg.append((lax.shift_right_logical(w, 9) & 0x7F80) + iota)
            row_ref = rows_s.at[i]
            for c in range(D // L):
                acc = plsc.load_gather(style_v, [vg[0] + c * L])
                for g in range(1, G):
                    acc = acc + plsc.load_gather(style_v, [vg[g] + c * L])
                row_ref[pl.ds(c * L, L)] = acc

        gathers[j].wait()
        writes[j] = pltpu.async_copy(
            t_bufs[j % 2],
            out_hbm.at[pl.ds(base + j * 128, 128), pl.ds(0, D)], semw)
        pltpu.async_copy(
            rows_s.at[pl.ds(j * 128, 128), :],
            out_hbm.at[pl.ds(base + j * 128, 128), pl.ds(D, D)], sems)

    writes[NCHUNK - 1].wait()
    for j in range(NCHUNK):
        pltpu.make_async_copy(
            rows_s.at[pl.ds(j * 128, 128), :],
            out_hbm.at[pl.ds(base + j * 128, 128), pl.ds(D, D)], sems).wait()


_sc_call = functools.partial(
    pl.kernel,
    out_type=jax.ShapeDtypeStruct((B, 2 * D), jnp.float32),
    mesh=plsc.VectorSubcoreMesh(core_axis_name="c", subcore_axis_name="s"),
    compiler_params=pltpu.CompilerParams(
        needs_layout_passes=False, use_tc_tiling_on_sc=False),
    scratch_types=[
        pltpu.VMEM((NCHUNK, 128), jnp.int32),
        pltpu.VMEM((BPW * G // 2,), jnp.int32),
        pltpu.VMEM((N_GENRES * D,), jnp.float32),
        pltpu.VMEM((BPW, D), jnp.float32),
        pltpu.VMEM((128, D), jnp.float32),
        pltpu.VMEM((128, D), jnp.float32),
        pltpu.SemaphoreType.DMA,
        pltpu.SemaphoreType.DMA,
        pltpu.SemaphoreType.DMA,
        pltpu.SemaphoreType.DMA,
    ],
)(_sc_body)


def kernel(t, length, genres, t_table, style_table):
    del length
    t_idx = t.reshape(B // 128, 128)
    genres_flat = lax.bitcast_convert_type(
        genres.astype(jnp.int16).reshape(B, G // 2, 2), jnp.int32).reshape(-1)
    style_flat = style_table.reshape(-1)
    out = _sc_call(t_idx, genres_flat, t_table, style_flat)
    return out.reshape(B, 2 * D, 1)


# speedup vs baseline: 1.3424x; 1.3424x over previous
import functools

import jax
import jax.numpy as jnp
from jax import lax
from jax.experimental import pallas as pl
from jax.experimental.pallas import tpu as pltpu, tpu_sc as plsc

B = 16384
D = 128
G = 8
N_GENRES = 100
T_ROWS = 1001

NC = 2
NS = 16
NW = NC * NS
BPW = B // NW
L = 16
NCHUNK = BPW // 128
GRPS = 128 // L


def _sc_body(t_idx_hbm, genres_hbm, t_table_hbm, style_hbm, out_hbm,
             idx_v, gv_v, style_v, rows_s, t_buf0, t_buf1,
             sem0, sem1, semw, sems):
    wid = lax.axis_index("s") * NC + lax.axis_index("c")
    base = wid * BPW

    pltpu.sync_copy(t_idx_hbm.at[pl.ds(wid * NCHUNK, NCHUNK), :], idx_v)
    pltpu.sync_copy(genres_hbm.at[pl.ds(base * G, BPW * G)], gv_v)
    pltpu.sync_copy(style_hbm, style_v)

    t_bufs = [t_buf0, t_buf1]
    gsems = [sem0, sem1]
    iota = lax.iota(jnp.int32, L)

    gathers = [None] * NCHUNK
    writes = [None] * NCHUNK
    gathers[0] = pltpu.async_copy(t_table_hbm.at[idx_v.at[0]], t_buf0, sem0)

    for j in range(NCHUNK):
        if j >= 1:
            writes[j - 1].wait()
        if j + 1 < NCHUNK:
            gathers[j + 1] = pltpu.async_copy(
                t_table_hbm.at[idx_v.at[j + 1]], t_bufs[(j + 1) % 2],
                gsems[(j + 1) % 2])

        @plsc.parallel_loop(j * 128, (j + 1) * 128, unroll=1)
        def _elem(i):
            sb = jnp.full((L,), i * G, jnp.int32)
            gvals = [plsc.load_gather(gv_v, [sb + g]) for g in range(G)]
            vg = [(gvals[g] << 7) + iota for g in range(G)]
            row_ref = rows_s.at[i]
            for c in range(D // L):
                acc = plsc.load_gather(style_v, [vg[0] + c * L])
                for g in range(1, G):
                    acc = acc + plsc.load_gather(style_v, [vg[g] + c * L])
                row_ref[pl.ds(c * L, L)] = acc

        gathers[j].wait()
        writes[j] = pltpu.async_copy(
            t_bufs[j % 2],
            out_hbm.at[pl.ds(base + j * 128, 128), pl.ds(0, D)], semw)
        pltpu.async_copy(
            rows_s.at[pl.ds(j * 128, 128), :],
            out_hbm.at[pl.ds(base + j * 128, 128), pl.ds(D, D)], sems)

    writes[NCHUNK - 1].wait()
    for j in range(NCHUNK):
        pltpu.make_async_copy(
            rows_s.at[pl.ds(j * 128, 128), :],
            out_hbm.at[pl.ds(base + j * 128, 128), pl.ds(D, D)], sems).wait()


_sc_call = functools.partial(
    pl.kernel,
    out_type=jax.ShapeDtypeStruct((B, 2 * D), jnp.float32),
    mesh=plsc.VectorSubcoreMesh(core_axis_name="c", subcore_axis_name="s"),
    compiler_params=pltpu.CompilerParams(
        needs_layout_passes=False, use_tc_tiling_on_sc=False,
        disable_bounds_checks=True, skip_device_barrier=True),
    scratch_types=[
        pltpu.VMEM((NCHUNK, 128), jnp.int32),
        pltpu.VMEM((BPW * G,), jnp.int32),
        pltpu.VMEM((N_GENRES * D,), jnp.float32),
        pltpu.VMEM((BPW, D), jnp.float32),
        pltpu.VMEM((128, D), jnp.float32),
        pltpu.VMEM((128, D), jnp.float32),
        pltpu.SemaphoreType.DMA,
        pltpu.SemaphoreType.DMA,
        pltpu.SemaphoreType.DMA,
        pltpu.SemaphoreType.DMA,
    ],
)(_sc_body)


def kernel(t, length, genres, t_table, style_table):
    del length
    t_idx = t.reshape(B // 128, 128)
    genres_flat = genres.reshape(-1)
    style_flat = style_table.reshape(-1)
    out = _sc_call(t_idx, genres_flat, t_table, style_flat)
    return out.reshape(B, 2 * D, 1)


# speedup vs baseline: 1.3425x; 1.0001x over previous
import functools

import jax
import jax.numpy as jnp
from jax import lax
from jax.experimental import pallas as pl
from jax.experimental.pallas import tpu as pltpu, tpu_sc as plsc

B = 16384
D = 128
G = 8
N_GENRES = 100
T_ROWS = 1001

NC = 2
NS = 16
NW = NC * NS
BPW = B // NW
L = 16
NCHUNK = BPW // 128
GRPS = 128 // L


def _sc_body(t_idx_hbm, genres_hbm, t_table_hbm, style_hbm, out_hbm,
             idx_v, gv_v, style_v, rows_s, t_buf0, t_buf1,
             sem0, sem1, semw, sems):
    wid = lax.axis_index("s") * NC + lax.axis_index("c")
    base = wid * BPW

    pltpu.sync_copy(t_idx_hbm.at[pl.ds(wid * NCHUNK, NCHUNK), :], idx_v)
    pltpu.sync_copy(genres_hbm.at[pl.ds(base * G, BPW * G)], gv_v)
    pltpu.sync_copy(style_hbm, style_v)

    t_bufs = [t_buf0, t_buf1]
    gsems = [sem0, sem1]
    iota = lax.iota(jnp.int32, L)

    gathers = [None] * NCHUNK
    writes = [None] * NCHUNK
    gathers[0] = pltpu.async_copy(t_table_hbm.at[idx_v.at[0]], t_buf0, sem0)

    for j in range(NCHUNK):
        if j >= 1:
            writes[j - 1].wait()
        if j + 1 < NCHUNK:
            gathers[j + 1] = pltpu.async_copy(
                t_table_hbm.at[idx_v.at[j + 1]], t_bufs[(j + 1) % 2],
                gsems[(j + 1) % 2])

        @plsc.parallel_loop(j * 128, (j + 1) * 128, unroll=1)
        def _elem(i):
            sb = jnp.full((L,), i * G, jnp.int32)
            gvals = [plsc.load_gather(gv_v, [sb + g]) for g in range(G)]
            vg = [(gvals[g] << 7) + iota for g in range(G)]
            row_ref = rows_s.at[i]
            for c in range(D // L):
                acc = plsc.load_gather(style_v, [vg[0] + c * L])
                for g in range(1, G):
                    acc = acc + plsc.load_gather(style_v, [vg[g] + c * L])
                row_ref[pl.ds(c * L, L)] = acc

        gathers[j].wait()
        writes[j] = pltpu.async_copy(
            t_bufs[j % 2],
            out_hbm.at[pl.ds(base + j * 128, 128), pl.ds(0, D)], semw)
        pltpu.async_copy(
            rows_s.at[pl.ds(j * 128, 128), :],
            out_hbm.at[pl.ds(base + j * 128, 128), pl.ds(D, D)], sems)

    writes[NCHUNK - 1].wait()
    for j in range(NCHUNK):
        pltpu.make_async_copy(
            rows_s.at[pl.ds(j * 128, 128), :],
            out_hbm.at[pl.ds(base + j * 128, 128), pl.ds(D, D)], sems).wait()


_sc_call = functools.partial(
    pl.kernel,
    out_type=jax.ShapeDtypeStruct((B, 2 * D), jnp.float32),
    mesh=plsc.VectorSubcoreMesh(core_axis_name="c", subcore_axis_name="s"),
    compiler_params=pltpu.CompilerParams(
        needs_layout_passes=False, use_tc_tiling_on_sc=False),
    scratch_types=[
        pltpu.VMEM((NCHUNK, 128), jnp.int32),
        pltpu.VMEM((BPW * G,), jnp.int32),
        pltpu.VMEM((N_GENRES * D,), jnp.float32),
        pltpu.VMEM((BPW, D), jnp.float32),
        pltpu.VMEM((128, D), jnp.float32),
        pltpu.VMEM((128, D), jnp.float32),
        pltpu.SemaphoreType.DMA,
        pltpu.SemaphoreType.DMA,
        pltpu.SemaphoreType.DMA,
        pltpu.SemaphoreType.DMA,
    ],
)(_sc_body)


def kernel(t, length, genres, t_table, style_table):
    del length
    t_idx = t.reshape(B // 128, 128)
    genres_flat = genres.reshape(-1)
    style_flat = style_table.reshape(-1)
    out = _sc_call(t_idx, genres_flat, t_table, style_flat)
    return out.reshape(B, 2 * D, 1)
